# baseline (speedup 1.0000x reference)
import dataclasses

import jax
import jax.numpy as jnp
from jax import lax
from jax.experimental import pallas as pl
from jax.experimental.pallas import tpu as pltpu
from jax.experimental.pallas import tpu_sc as plsc


def _sc_compiler_params():
    cp = pltpu.CompilerParams(use_tc_tiling_on_sc=False)
    if "needs_layout_passes" in pltpu.CompilerParams.__dataclass_fields__:
        cp = dataclasses.replace(cp, needs_layout_passes=False)
    return cp


_V = 100000
_E = 32
_L = 16
_C = 8192


def kernel(idx, directions):
    b, s = idx.shape
    assert directions.shape == (_V, _E)
    assert b % _C == 0
    nq = b // _C
    nt = s * nq

    table_flat = directions.T.reshape(_V * _E)
    idx_flat = idx.T.reshape(b * s)

    mesh = plsc.VectorSubcoreMesh(core_axis_name="core", subcore_axis_name="subcore")

    @pl.kernel(out_type=jax.ShapeDtypeStruct((s, _E // 8, b // 128, 1024),
                                             directions.dtype),
               mesh=mesh,
               scratch_types=[
                   pltpu.VMEM((_V,), jnp.float32),
                   pltpu.VMEM((_C,), jnp.int32),
                   pltpu.VMEM((2, 64, 128), jnp.float32),
                   pltpu.VMEM_SHARED((2, _C), jnp.int32),
                   pltpu.SemaphoreType.DMA,
                   pltpu.SemaphoreType.DMA,
                   pltpu.SemaphoreType.DMA,
               ],
               compiler_params=_sc_compiler_params())
    def gather_kernel(table_hbm, idx_hbm, out_hbm, row_v, idx_v, out_v,
                      idx_sp, psem, osem0, osem1):
        sub = lax.axis_index("subcore")
        e = lax.axis_index("core") * 16 + sub
        e_hi = e // 8
        e_lo = e % 8
        pltpu.sync_copy(table_hbm.at[pl.ds(e * _V, _V)], row_v)

        @pl.when(sub == 0)
        def _():
            pltpu.async_copy(idx_hbm.at[pl.ds(0, _C)], idx_sp.at[0], psem).wait()

        plsc.subcore_barrier()

        def step(t, par):
            nxt = jnp.minimum(t + 1, nt - 1)

            @pl.when(sub == 0)
            def _():
                pltpu.async_copy(idx_hbm.at[pl.ds(nxt * _C, _C)],
                                 idx_sp.at[1 - par], psem)

            pltpu.sync_copy(idx_sp.at[par], idx_v)

            osem = osem0 if par == 0 else osem1

            @pl.when(t >= 2)
            def _():
                pltpu.make_async_copy(
                    out_hbm.at[0, 0, pl.ds(0, 64), pl.ds(0, 128)],
                    out_v.at[par], osem).wait()

            @plsc.parallel_loop(0, _C, step=_L, unroll=24)
            def _(c):
                ids = idx_v[pl.ds(c, _L)]
                out_v[par, c // 128, pl.ds(c % 128, _L)] = \
                    plsc.load_gather(row_v, [ids])

            si = t // nq
            q = t % nq
            pltpu.async_copy(
                out_v.at[par],
                out_hbm.at[si, e_hi, pl.ds(q * 64, 64), pl.ds(e_lo * 128, 128)],
                osem)

            @pl.when(sub == 0)
            def _():
                pltpu.make_async_copy(idx_hbm.at[pl.ds(nxt * _C, _C)],
                                      idx_sp.at[1 - par], psem).wait()

            plsc.subcore_barrier()

        @pl.loop(0, nt, step=2)
        def _(t0):
            step(t0, 0)
            step(t0 + 1, 1)

        pltpu.make_async_copy(out_hbm.at[0, 0, pl.ds(0, 64), pl.ds(0, 128)],
                              out_v.at[0], osem0).wait()
        pltpu.make_async_copy(out_hbm.at[0, 0, pl.ds(0, 64), pl.ds(0, 128)],
                              out_v.at[1], osem1).wait()

    out5 = gather_kernel(table_flat, idx_flat)
    return (out5.reshape(s, _E // 8, b // 128, 8, 128)
            .transpose(2, 4, 0, 1, 3)
            .reshape(b, s, _E))


# speedup vs baseline: 1.0283x; 1.0283x over previous
import dataclasses

import jax
import jax.numpy as jnp
from jax import lax
from jax.experimental import pallas as pl
from jax.experimental.pallas import tpu as pltpu
from jax.experimental.pallas import tpu_sc as plsc


def _sc_compiler_params():
    cp = pltpu.CompilerParams(use_tc_tiling_on_sc=False)
    if "needs_layout_passes" in pltpu.CompilerParams.__dataclass_fields__:
        cp = dataclasses.replace(cp, needs_layout_passes=False)
    return cp


_V = 100000
_E = 32
_L = 16
_C = 8192


def kernel(idx, directions):
    b, s = idx.shape
    assert directions.shape == (_V, _E)
    assert b % _C == 0
    nq = b // _C
    nt = s * nq

    table_flat = directions.T.reshape(_V * _E)
    idx_flat = idx.T.reshape(b * s)

    mesh = plsc.VectorSubcoreMesh(core_axis_name="core", subcore_axis_name="subcore")

    @pl.kernel(out_type=jax.ShapeDtypeStruct((s, _E // 8, b // 128, 1024),
                                             directions.dtype),
               mesh=mesh,
               scratch_types=[
                   pltpu.VMEM((_V,), jnp.float32),
                   pltpu.VMEM((_C,), jnp.int32),
                   pltpu.VMEM((2, 64, 128), jnp.float32),
                   pltpu.VMEM_SHARED((2, _C), jnp.int32),
                   pltpu.SemaphoreType.DMA,
                   pltpu.SemaphoreType.DMA,
                   pltpu.SemaphoreType.DMA,
                   pltpu.SemaphoreType.DMA,
                   pltpu.SemaphoreType.DMA,
               ],
               compiler_params=_sc_compiler_params())
    def gather_kernel(table_hbm, idx_hbm, out_hbm, row_v, idx_v, out_v,
                      idx_sp, psem, osem0, osem1, csem0, csem1):
        sub = lax.axis_index("subcore")
        e = lax.axis_index("core") * 16 + sub
        e_hi = e // 8
        e_lo = e % 8
        pltpu.sync_copy(table_hbm.at[pl.ds(e * _V, _V)], row_v)

        @pl.when(sub == 0)
        def _():
            pltpu.async_copy(idx_hbm.at[pl.ds(0, _C)], idx_sp.at[0], psem).wait()

        plsc.subcore_barrier()

        def step(t, par):
            nxt = jnp.minimum(t + 1, nt - 1)

            @pl.when(sub == 0)
            def _():
                pltpu.async_copy(idx_hbm.at[pl.ds(nxt * _C, _C)],
                                 idx_sp.at[1 - par], psem)

            h = _C // 2
            pltpu.async_copy(idx_sp.at[par, pl.ds(0, h)],
                             idx_v.at[pl.ds(0, h)], csem0)
            pltpu.async_copy(idx_sp.at[par, pl.ds(h, h)],
                             idx_v.at[pl.ds(h, h)], csem1)

            osem = osem0 if par == 0 else osem1

            @pl.when(t >= 2)
            def _():
                pltpu.make_async_copy(
                    out_hbm.at[0, 0, pl.ds(0, 64), pl.ds(0, 128)],
                    out_v.at[par], osem).wait()

            h = _C // 2
            pltpu.make_async_copy(idx_hbm.at[pl.ds(0, h)],
                                  idx_v.at[pl.ds(0, h)], csem0).wait()

            @plsc.parallel_loop(0, h, step=_L, unroll=16)
            def _(c):
                ids = idx_v[pl.ds(c, _L)]
                out_v[par, c // 128, pl.ds(c % 128, _L)] = \
                    plsc.load_gather(row_v, [ids])

            pltpu.make_async_copy(idx_hbm.at[pl.ds(0, h)],
                                  idx_v.at[pl.ds(h, h)], csem1).wait()

            @plsc.parallel_loop(h, _C, step=_L, unroll=16)
            def _(c):
                ids = idx_v[pl.ds(c, _L)]
                out_v[par, c // 128, pl.ds(c % 128, _L)] = \
                    plsc.load_gather(row_v, [ids])

            si = t // nq
            q = t % nq
            pltpu.async_copy(
                out_v.at[par],
                out_hbm.at[si, e_hi, pl.ds(q * 64, 64), pl.ds(e_lo * 128, 128)],
                osem)

            @pl.when(sub == 0)
            def _():
                pltpu.make_async_copy(idx_hbm.at[pl.ds(nxt * _C, _C)],
                                      idx_sp.at[1 - par], psem).wait()

            plsc.subcore_barrier()

        @pl.loop(0, nt, step=2)
        def _(t0):
            step(t0, 0)
            step(t0 + 1, 1)

        pltpu.make_async_copy(out_hbm.at[0, 0, pl.ds(0, 64), pl.ds(0, 128)],
                              out_v.at[0], osem0).wait()
        pltpu.make_async_copy(out_hbm.at[0, 0, pl.ds(0, 64), pl.ds(0, 128)],
                              out_v.at[1], osem1).wait()

    out5 = gather_kernel(table_flat, idx_flat)
    return (out5.reshape(s, _E // 8, b // 128, 8, 128)
            .transpose(2, 4, 0, 1, 3)
            .reshape(b, s, _E))


# speedup vs baseline: 1.0349x; 1.0065x over previous
import dataclasses

import jax
import jax.numpy as jnp
from jax import lax
from jax.experimental import pallas as pl
from jax.experimental.pallas import tpu as pltpu
from jax.experimental.pallas import tpu_sc as plsc


def _sc_compiler_params():
    cp = pltpu.CompilerParams(use_tc_tiling_on_sc=False)
    if "needs_layout_passes" in pltpu.CompilerParams.__dataclass_fields__:
        cp = dataclasses.replace(cp, needs_layout_passes=False)
    return cp


_V = 100000
_E = 32
_L = 16
_C = 8192


def kernel(idx, directions):
    b, s = idx.shape
    assert directions.shape == (_V, _E)
    assert b % _C == 0
    nq = b // _C
    nt = s * nq

    table_flat = directions.T.reshape(_V * _E)
    idx_flat = idx.T.reshape(b * s)

    mesh = plsc.VectorSubcoreMesh(core_axis_name="core", subcore_axis_name="subcore")

    @pl.kernel(out_type=jax.ShapeDtypeStruct((s, _E // 8, b // 128, 1024),
                                             directions.dtype),
               mesh=mesh,
               scratch_types=[
                   pltpu.VMEM((_V,), jnp.float32),
                   pltpu.VMEM((_C,), jnp.int32),
                   pltpu.VMEM((2, 64, 128), jnp.float32),
                   pltpu.VMEM_SHARED((2, _C), jnp.int32),
                   pltpu.SemaphoreType.DMA,
                   pltpu.SemaphoreType.DMA,
                   pltpu.SemaphoreType.DMA,
               ],
               compiler_params=_sc_compiler_params())
    def gather_kernel(table_hbm, idx_hbm, out_hbm, row_v, idx_v, out_v,
                      idx_sp, psem, osem0, osem1):
        sub = lax.axis_index("subcore")
        e = lax.axis_index("core") * 16 + sub
        e_hi = e // 8
        e_lo = e % 8
        pltpu.sync_copy(table_hbm.at[pl.ds(e * _V, _V)], row_v)

        @pl.when(sub == 0)
        def _():
            pltpu.async_copy(idx_hbm.at[pl.ds(0, _C)], idx_sp.at[0], psem).wait()

        plsc.subcore_barrier()

        def step(t, par):
            nxt = jnp.minimum(t + 1, nt - 1)

            @pl.when(sub == 0)
            def _():
                pltpu.async_copy(idx_hbm.at[pl.ds(nxt * _C, _C)],
                                 idx_sp.at[1 - par], psem)

            pltpu.sync_copy(idx_sp.at[par], idx_v)

            osem = osem0 if par == 0 else osem1

            @pl.when(t >= 2)
            def _():
                pltpu.make_async_copy(
                    out_hbm.at[0, 0, pl.ds(0, 64), pl.ds(0, 128)],
                    out_v.at[par], osem).wait()

            @plsc.parallel_loop(0, _C, step=_L, unroll=16)
            def _(c):
                ids = idx_v[pl.ds(c, _L)]
                out_v[par, c // 128, pl.ds(c % 128, _L)] = \
                    plsc.load_gather(row_v, [ids])

            si = t // nq
            q = t % nq
            pltpu.async_copy(
                out_v.at[par],
                out_hbm.at[si, e_hi, pl.ds(q * 64, 64), pl.ds(e_lo * 128, 128)],
                osem)

            @pl.when(sub == 0)
            def _():
                pltpu.make_async_copy(idx_hbm.at[pl.ds(nxt * _C, _C)],
                                      idx_sp.at[1 - par], psem).wait()

            plsc.subcore_barrier()

        @pl.loop(0, nt, step=2)
        def _(t0):
            step(t0, 0)
            step(t0 + 1, 1)

        pltpu.make_async_copy(out_hbm.at[0, 0, pl.ds(0, 64), pl.ds(0, 128)],
                              out_v.at[0], osem0).wait()
        pltpu.make_async_copy(out_hbm.at[0, 0, pl.ds(0, 64), pl.ds(0, 128)],
                              out_v.at[1], osem1).wait()

    out5 = gather_kernel(table_flat, idx_flat)
    return (out5.reshape(s, _E // 8, b // 128, 8, 128)
            .transpose(2, 4, 0, 1, 3)
            .reshape(b, s, _E))
